# baseline (speedup 1.0000x reference)
import functools

import jax
import jax.numpy as jnp
from jax import lax
from jax.experimental import pallas as pl
from jax.experimental.pallas import tpu as pltpu
from jax.experimental.pallas import tpu_sc as plsc


def _sc_reduce_zl(z_L):
    b_sz, s_seq, ld = z_L.shape
    info = plsc.get_sparse_core_info()
    nc, ns, lanes = info.num_cores, info.num_subcores, info.num_lanes
    nw = nc * ns
    cols = 128
    n_tiles = ld // cols
    splits = max(1, nw // n_tiles)
    rows_pw = s_seq // splits
    cg = cols // lanes
    rch = min(256, rows_pw)
    n_ch = rows_pw // rch
    chunks = [(b, j * rch) for b in range(b_sz) for j in range(n_ch)]

    @functools.partial(
        pl.kernel,
        mesh=plsc.VectorSubcoreMesh(core_axis_name="c", subcore_axis_name="s"),
        out_type=jax.ShapeDtypeStruct((splits, b_sz, ld), jnp.float32),
        scratch_types=[
            pltpu.VMEM((2, rch, cols), jnp.float32),
            pltpu.VMEM((b_sz, cols), jnp.float32),
            pltpu.SemaphoreType.DMA((2,)),
        ],
    )
    def k(zl_hbm, out_hbm, buf_ref, acc_ref, sems):
        wid = lax.axis_index("s") * nc + lax.axis_index("c")
        tile = wid // splits
        half = wid % splits
        c0 = tile * cols
        r0 = half * rows_pw

        def src(i):
            b, s0 = chunks[i]
            return zl_hbm.at[b, pl.ds(r0 + s0, rch), pl.ds(c0, cols)]

        pending = [pltpu.async_copy(src(0), buf_ref.at[0], sems.at[0])]
        for i in range(len(chunks)):
            p = i % 2
            if i + 1 < len(chunks):
                pending.append(pltpu.async_copy(
                    src(i + 1), buf_ref.at[(i + 1) % 2], sems.at[(i + 1) % 2]))
            pending[i].wait()
            b, s0 = chunks[i]

            def row_body(r, carry):
                return tuple(
                    carry[c] + buf_ref[p, r, pl.ds(c * lanes, lanes)]
                    for c in range(cg))

            zero = jnp.zeros((lanes,), jnp.float32)
            acc = lax.fori_loop(0, rch, row_body, (zero,) * cg)
            for c in range(cg):
                sl = pl.ds(c * lanes, lanes)
                if s0 == 0:
                    acc_ref[b, sl] = acc[c]
                else:
                    acc_ref[b, sl] += acc[c]
        for b in range(b_sz):
            pltpu.sync_copy(acc_ref.at[b],
                            out_hbm.at[half, b, pl.ds(c0, cols)])

    return k(z_L)


def _tc_body(zh1_ref, zh2_ref, sl_ref, wh_ref, wl_ref, keys_ref,
             w1_ref, b1_ref, w2_ref, b2_ref, conf_ref, idx_ref,
             acch_ref, *, s_seq, n_red):
    t = pl.program_id(0)

    @pl.when(t == 0)
    def _():
        acch_ref[...] = jnp.zeros_like(acch_ref)

    @pl.when(t < n_red)
    def _():
        n_c = n_red // acch_ref.shape[0]
        b = t // n_c
        acch_ref[pl.ds(b, 1), :] += (jnp.sum(zh1_ref[...], axis=1) +
                                     jnp.sum(zh2_ref[...], axis=1))

    @pl.when(t == n_red)
    def _():
        f32 = jnp.float32
        hi = lax.Precision.HIGHEST
        zbar_h = acch_ref[...] * (1.0 / s_seq)
        zbar_l = jnp.sum(sl_ref[...], axis=0) * (1.0 / s_seq)
        dn = (((1,), (1,)), ((), ()))
        hp = lax.dot_general(zbar_h, wh_ref[...], dn, precision=hi,
                             preferred_element_type=f32)
        lp = lax.dot_general(zbar_l, wl_ref[...], dn, precision=hi,
                             preferred_element_type=f32)
        kn = jnp.sqrt(jnp.sum(hp * hp, axis=1) + jnp.sum(lp * lp, axis=1))
        keys = keys_ref[...]
        pd = hp.shape[1]
        cn = jnp.sqrt(jnp.sum(keys * keys, axis=1))
        dots = (lax.dot_general(hp, keys[:, :pd], dn, precision=hi,
                                preferred_element_type=f32) +
                lax.dot_general(lp, keys[:, pd:], dn, precision=hi,
                                preferred_element_type=f32))
        sim = dots / jnp.maximum(kn[:, None] * cn[None, :], 1e-8)
        max_sim = jnp.max(sim, axis=1)
        k = sim.shape[1]
        iota = lax.broadcasted_iota(jnp.int32, sim.shape, 1)
        idx = jnp.min(jnp.where(sim == max_sim[:, None], iota, k), axis=1)
        w1 = w1_ref[...]
        h = (lax.dot_general(hp, w1[:, :pd], dn, precision=hi,
                             preferred_element_type=f32) +
             lax.dot_general(lp, w1[:, pd:2 * pd], dn, precision=hi,
                             preferred_element_type=f32) +
             lax.dot_general(max_sim[:, None], w1[:, 2 * pd:], dn,
                             precision=hi, preferred_element_type=f32) +
             b1_ref[...][None, :])
        h = 0.5 * h * (1.0 + lax.erf(h * (2.0 ** -0.5)))
        logit = jnp.sum(h * w2_ref[...], axis=1) + b2_ref[0]
        conf_ref[...] = jax.nn.sigmoid(logit)
        idx_ref[...] = idx.astype(jnp.int32)


def _bcast_body(idx_ref, code_hbm, out_hbm, rows_ref, buf_ref, sem_rows,
                sem_w, *, rep, n_wr):
    b_sz = rows_ref.shape[0]
    ld = rows_ref.shape[2]
    for b in range(b_sz):
        pltpu.make_async_copy(
            code_hbm.at[pl.ds(idx_ref[b], 1), :],
            rows_ref.at[b], sem_rows.at[b]).start()
    for b in range(b_sz):
        pltpu.make_async_copy(
            code_hbm.at[pl.ds(idx_ref[b], 1), :],
            rows_ref.at[b], sem_rows.at[b]).wait()
    for b in range(b_sz):
        buf_ref[b, :, :] = jnp.broadcast_to(rows_ref[b], (rep, ld))
    for b in range(b_sz):
        for j in range(n_wr):
            pltpu.make_async_copy(
                buf_ref.at[b],
                out_hbm.at[b, pl.ds(j * rep, rep), :],
                sem_w.at[b * n_wr + j]).start()
    for b in range(b_sz):
        for j in range(n_wr):
            pltpu.make_async_copy(
                buf_ref.at[b],
                out_hbm.at[b, pl.ds(j * rep, rep), :],
                sem_w.at[b * n_wr + j]).wait()


def kernel(z_H, z_L, W_h, W_l, codebook, codebook_keys, W1, b1, W2, b2):
    b_sz, s_seq, hd = z_H.shape
    ld = z_L.shape[2]

    sums_l = _sc_reduce_zl(z_L)

    s_half = s_seq // 2
    red_ch = min(512, s_half)
    n_c = s_half // red_ch
    n_red = b_sz * n_c

    def idx_lo(t):
        tc = jnp.minimum(t, n_red - 1)
        return (tc // n_c, tc % n_c, 0)

    def idx_hi(t):
        tc = jnp.minimum(t, n_red - 1)
        return (tc // n_c, n_c + tc % n_c, 0)

    conf, idx = pl.pallas_call(
        functools.partial(_tc_body, s_seq=s_seq, n_red=n_red),
        grid=(n_red + 1,),
        in_specs=[
            pl.BlockSpec((1, red_ch, hd), idx_lo),
            pl.BlockSpec((1, red_ch, hd), idx_hi),
            pl.BlockSpec(sums_l.shape, lambda t: (0, 0, 0)),
            pl.BlockSpec((W_h.shape[0], hd), lambda t: (0, 0)),
            pl.BlockSpec((W_l.shape[0], ld), lambda t: (0, 0)),
            pl.BlockSpec(codebook_keys.shape, lambda t: (0, 0)),
            pl.BlockSpec(W1.shape, lambda t: (0, 0)),
            pl.BlockSpec(b1.shape, lambda t: (0,)),
            pl.BlockSpec(W2.shape, lambda t: (0, 0)),
            pl.BlockSpec(b2.shape, lambda t: (0,)),
        ],
        out_specs=[
            pl.BlockSpec((b_sz,), lambda t: (0,)),
            pl.BlockSpec((b_sz,), lambda t: (0,)),
        ],
        out_shape=[
            jax.ShapeDtypeStruct((b_sz,), jnp.float32),
            jax.ShapeDtypeStruct((b_sz,), jnp.int32),
        ],
        scratch_shapes=[
            pltpu.VMEM((b_sz, hd), jnp.float32),
        ],
    )(z_H, z_H, sums_l, W_h, W_l, codebook_keys, W1, b1, W2, b2)

    rep = min(256, s_seq)
    n_wr = s_seq // rep
    nearest_code = pl.pallas_call(
        functools.partial(_bcast_body, rep=rep, n_wr=n_wr),
        grid_spec=pltpu.PrefetchScalarGridSpec(
            num_scalar_prefetch=1,
            grid=(1,),
            in_specs=[pl.BlockSpec(memory_space=pl.ANY)],
            out_specs=pl.BlockSpec(memory_space=pl.ANY),
            scratch_shapes=[
                pltpu.VMEM((b_sz, 1, ld), jnp.float32),
                pltpu.VMEM((b_sz, rep, ld), jnp.float32),
                pltpu.SemaphoreType.DMA((b_sz,)),
                pltpu.SemaphoreType.DMA((b_sz * n_wr,)),
            ],
        ),
        out_shape=jax.ShapeDtypeStruct((b_sz, s_seq, ld), jnp.float32),
    )(idx, codebook)

    return conf, nearest_code, idx


# speedup vs baseline: 1.0912x; 1.0912x over previous
import functools

import jax
import jax.numpy as jnp
from jax import lax
from jax.experimental import pallas as pl
from jax.experimental.pallas import tpu as pltpu
from jax.experimental.pallas import tpu_sc as plsc


def _sc_reduce_zl(z_L):
    b_sz, s_seq, ld = z_L.shape
    info = plsc.get_sparse_core_info()
    nc, ns, lanes = info.num_cores, info.num_subcores, info.num_lanes
    nw = nc * ns
    cols = 128
    n_tiles = ld // cols
    splits = max(1, nw // n_tiles)
    rows_pw = s_seq // splits
    cg = cols // lanes
    rch = min(256, rows_pw)
    n_ch = rows_pw // rch
    chunks = [(b, j * rch) for b in range(b_sz) for j in range(n_ch)]

    @functools.partial(
        pl.kernel,
        mesh=plsc.VectorSubcoreMesh(core_axis_name="c", subcore_axis_name="s"),
        out_type=jax.ShapeDtypeStruct((splits, b_sz, ld), jnp.float32),
        scratch_types=[
            pltpu.VMEM((2, rch, cols), jnp.float32),
            pltpu.VMEM((b_sz, cols), jnp.float32),
            pltpu.SemaphoreType.DMA((2,)),
        ],
    )
    def k(zl_hbm, out_hbm, buf_ref, acc_ref, sems):
        wid = lax.axis_index("s") * nc + lax.axis_index("c")
        tile = wid // splits
        half = wid % splits
        c0 = tile * cols
        r0 = half * rows_pw

        def src(i):
            b, s0 = chunks[i]
            return zl_hbm.at[b, pl.ds(r0 + s0, rch), pl.ds(c0, cols)]

        pending = [pltpu.async_copy(src(0), buf_ref.at[0], sems.at[0])]
        for i in range(len(chunks)):
            p = i % 2
            if i + 1 < len(chunks):
                pending.append(pltpu.async_copy(
                    src(i + 1), buf_ref.at[(i + 1) % 2], sems.at[(i + 1) % 2]))
            pending[i].wait()
            b, s0 = chunks[i]

            def row_body(r, carry):
                return tuple(
                    carry[c] + buf_ref[p, r, pl.ds(c * lanes, lanes)]
                    for c in range(cg))

            zero = jnp.zeros((lanes,), jnp.float32)
            acc = lax.fori_loop(0, rch, row_body, (zero,) * cg)
            for c in range(cg):
                sl = pl.ds(c * lanes, lanes)
                if s0 == 0:
                    acc_ref[b, sl] = acc[c]
                else:
                    acc_ref[b, sl] += acc[c]
        for b in range(b_sz):
            pltpu.sync_copy(acc_ref.at[b],
                            out_hbm.at[half, b, pl.ds(c0, cols)])

    return k(z_L)


def _tc_reduce_body(zh1_ref, zh2_ref, sh_ref, *, n_c):
    t = pl.program_id(0)

    @pl.when(t % n_c == 0)
    def _():
        sh_ref[...] = jnp.zeros_like(sh_ref)

    sh_ref[...] += (jnp.sum(zh1_ref[...], axis=1, keepdims=True) +
                    jnp.sum(zh2_ref[...], axis=1, keepdims=True))


def _head_body(sh_ref, sl_ref, wh_ref, wl_ref, keys_ref,
               w1_ref, b1_ref, w2_ref, b2_ref, conf_ref, idx_ref, *, s_seq):
    f32 = jnp.float32
    hi = lax.Precision.HIGHEST
    b_sz = sh_ref.shape[0]
    zbar_h = sh_ref[...].reshape(b_sz, -1) * (1.0 / s_seq)
    zbar_l = jnp.sum(sl_ref[...], axis=0) * (1.0 / s_seq)
    dn = (((1,), (1,)), ((), ()))
    hp = lax.dot_general(zbar_h, wh_ref[...], dn, precision=hi,
                         preferred_element_type=f32)
    lp = lax.dot_general(zbar_l, wl_ref[...], dn, precision=hi,
                         preferred_element_type=f32)
    kn = jnp.sqrt(jnp.sum(hp * hp, axis=1) + jnp.sum(lp * lp, axis=1))
    keys = keys_ref[...]
    pd = hp.shape[1]
    cn = jnp.sqrt(jnp.sum(keys * keys, axis=1))
    dots = (lax.dot_general(hp, keys[:, :pd], dn, precision=hi,
                            preferred_element_type=f32) +
            lax.dot_general(lp, keys[:, pd:], dn, precision=hi,
                            preferred_element_type=f32))
    sim = dots / jnp.maximum(kn[:, None] * cn[None, :], 1e-8)
    max_sim = jnp.max(sim, axis=1)
    k = sim.shape[1]
    iota = lax.broadcasted_iota(jnp.int32, sim.shape, 1)
    idx = jnp.min(jnp.where(sim == max_sim[:, None], iota, k), axis=1)
    w1 = w1_ref[...]
    h = (lax.dot_general(hp, w1[:, :pd], dn, precision=hi,
                         preferred_element_type=f32) +
         lax.dot_general(lp, w1[:, pd:2 * pd], dn, precision=hi,
                         preferred_element_type=f32) +
         lax.dot_general(max_sim[:, None], w1[:, 2 * pd:], dn,
                         precision=hi, preferred_element_type=f32) +
         b1_ref[...][None, :])
    h = 0.5 * h * (1.0 + lax.erf(h * (2.0 ** -0.5)))
    logit = jnp.sum(h * w2_ref[...], axis=1) + b2_ref[0]
    conf_ref[...] = jax.nn.sigmoid(logit)
    idx_ref[...] = idx.astype(jnp.int32)


def _bcast_body(idx_ref, code_hbm, out_hbm, rows_ref, buf_ref, sem_rows,
                sem_w, *, rep, n_wr):
    b_sz = rows_ref.shape[0]
    ld = rows_ref.shape[2]
    for b in range(b_sz):
        pltpu.make_async_copy(
            code_hbm.at[pl.ds(idx_ref[b], 1), :],
            rows_ref.at[b], sem_rows.at[b]).start()
    for b in range(b_sz):
        pltpu.make_async_copy(
            code_hbm.at[pl.ds(idx_ref[b], 1), :],
            rows_ref.at[b], sem_rows.at[b]).wait()
    for b in range(b_sz):
        buf_ref[b, :, :] = jnp.broadcast_to(rows_ref[b], (rep, ld))
    for b in range(b_sz):
        for j in range(n_wr):
            pltpu.make_async_copy(
                buf_ref.at[b],
                out_hbm.at[b, pl.ds(j * rep, rep), :],
                sem_w.at[b * n_wr + j]).start()
    for b in range(b_sz):
        for j in range(n_wr):
            pltpu.make_async_copy(
                buf_ref.at[b],
                out_hbm.at[b, pl.ds(j * rep, rep), :],
                sem_w.at[b * n_wr + j]).wait()


def kernel(z_H, z_L, W_h, W_l, codebook, codebook_keys, W1, b1, W2, b2):
    b_sz, s_seq, hd = z_H.shape
    ld = z_L.shape[2]

    sums_l = _sc_reduce_zl(z_L)

    s_half = s_seq // 2
    red_ch = min(512, s_half)
    n_c = s_half // red_ch
    n_red = b_sz * n_c

    def idx_lo(t):
        return (t // n_c, t % n_c, 0)

    def idx_hi(t):
        return (t // n_c, n_c + t % n_c, 0)

    sums_h = pl.pallas_call(
        functools.partial(_tc_reduce_body, n_c=n_c),
        grid=(n_red,),
        in_specs=[
            pl.BlockSpec((1, red_ch, hd), idx_lo),
            pl.BlockSpec((1, red_ch, hd), idx_hi),
        ],
        out_specs=pl.BlockSpec((1, 1, hd), lambda t: (t // n_c, 0, 0)),
        out_shape=jax.ShapeDtypeStruct((b_sz, 1, hd), jnp.float32),
    )(z_H, z_H)

    conf, idx = pl.pallas_call(
        functools.partial(_head_body, s_seq=s_seq),
        out_shape=[
            jax.ShapeDtypeStruct((b_sz,), jnp.float32),
            jax.ShapeDtypeStruct((b_sz,), jnp.int32),
        ],
    )(sums_h, sums_l, W_h, W_l, codebook_keys, W1, b1, W2, b2)

    rep = min(256, s_seq)
    n_wr = s_seq // rep
    nearest_code = pl.pallas_call(
        functools.partial(_bcast_body, rep=rep, n_wr=n_wr),
        grid_spec=pltpu.PrefetchScalarGridSpec(
            num_scalar_prefetch=1,
            grid=(1,),
            in_specs=[pl.BlockSpec(memory_space=pl.ANY)],
            out_specs=pl.BlockSpec(memory_space=pl.ANY),
            scratch_shapes=[
                pltpu.VMEM((b_sz, 1, ld), jnp.float32),
                pltpu.VMEM((b_sz, rep, ld), jnp.float32),
                pltpu.SemaphoreType.DMA((b_sz,)),
                pltpu.SemaphoreType.DMA((b_sz * n_wr,)),
            ],
        ),
        out_shape=jax.ShapeDtypeStruct((b_sz, s_seq, ld), jnp.float32),
    )(idx, codebook)

    return conf, nearest_code, idx


# speedup vs baseline: 1.3409x; 1.2288x over previous
import functools

import jax
import jax.numpy as jnp
from jax import lax
from jax.experimental import pallas as pl
from jax.experimental.pallas import tpu as pltpu


def _fused_body(zh1_ref, zh2_ref, zl1_ref, zl2_ref, wh_ref, wl_ref, keys_ref,
                w1_ref, b1_ref, w2_ref, b2_ref, conf_ref, idx_ref,
                acch_ref, accl_ref, *, s_seq, n_red):
    t = pl.program_id(0)

    @pl.when(t == 0)
    def _():
        acch_ref[...] = jnp.zeros_like(acch_ref)
        accl_ref[...] = jnp.zeros_like(accl_ref)

    @pl.when(t < n_red)
    def _():
        n_c = n_red // acch_ref.shape[0]
        b = t // n_c
        acch_ref[pl.ds(b, 1), :] += (jnp.sum(zh1_ref[...], axis=1) +
                                     jnp.sum(zh2_ref[...], axis=1))
        accl_ref[pl.ds(b, 1), :] += (jnp.sum(zl1_ref[...], axis=1) +
                                     jnp.sum(zl2_ref[...], axis=1))

    @pl.when(t == n_red)
    def _():
        f32 = jnp.float32
        hi = lax.Precision.HIGHEST
        zbar_h = acch_ref[...] * (1.0 / s_seq)
        zbar_l = accl_ref[...] * (1.0 / s_seq)
        dn = (((1,), (1,)), ((), ()))
        hp = lax.dot_general(zbar_h, wh_ref[...], dn, precision=hi,
                             preferred_element_type=f32)
        lp = lax.dot_general(zbar_l, wl_ref[...], dn, precision=hi,
                             preferred_element_type=f32)
        kn = jnp.sqrt(jnp.sum(hp * hp, axis=1) + jnp.sum(lp * lp, axis=1))
        keys = keys_ref[...]
        pd = hp.shape[1]
        cn = jnp.sqrt(jnp.sum(keys * keys, axis=1))
        dots = (lax.dot_general(hp, keys[:, :pd], dn, precision=hi,
                                preferred_element_type=f32) +
                lax.dot_general(lp, keys[:, pd:], dn, precision=hi,
                                preferred_element_type=f32))
        sim = dots / jnp.maximum(kn[:, None] * cn[None, :], 1e-8)
        max_sim = jnp.max(sim, axis=1)
        k = sim.shape[1]
        iota = lax.broadcasted_iota(jnp.int32, sim.shape, 1)
        idx = jnp.min(jnp.where(sim == max_sim[:, None], iota, k), axis=1)
        w1 = w1_ref[...]
        h = (lax.dot_general(hp, w1[:, :pd], dn, precision=hi,
                             preferred_element_type=f32) +
             lax.dot_general(lp, w1[:, pd:2 * pd], dn, precision=hi,
                             preferred_element_type=f32) +
             lax.dot_general(max_sim[:, None], w1[:, 2 * pd:], dn,
                             precision=hi, preferred_element_type=f32) +
             b1_ref[...][None, :])
        h = 0.5 * h * (1.0 + lax.erf(h * (2.0 ** -0.5)))
        logit = jnp.sum(h * w2_ref[...], axis=1) + b2_ref[0]
        conf_ref[...] = jax.nn.sigmoid(logit)
        idx_ref[...] = idx.astype(jnp.int32)


def _bcast_body(idx_ref, code_hbm, out_hbm, rows_ref, buf_ref, sem_rows,
                sem_w, *, rep, n_wr):
    b_sz = rows_ref.shape[0]
    ld = rows_ref.shape[2]
    for b in range(b_sz):
        pltpu.make_async_copy(
            code_hbm.at[pl.ds(idx_ref[b], 1), :],
            rows_ref.at[b], sem_rows.at[b]).start()
    for b in range(b_sz):
        pltpu.make_async_copy(
            code_hbm.at[pl.ds(idx_ref[b], 1), :],
            rows_ref.at[b], sem_rows.at[b]).wait()
    for b in range(b_sz):
        buf_ref[b, :, :] = jnp.broadcast_to(rows_ref[b], (rep, ld))
    for b in range(b_sz):
        for j in range(n_wr):
            pltpu.make_async_copy(
                buf_ref.at[b],
                out_hbm.at[b, pl.ds(j * rep, rep), :],
                sem_w.at[b * n_wr + j]).start()
    for b in range(b_sz):
        for j in range(n_wr):
            pltpu.make_async_copy(
                buf_ref.at[b],
                out_hbm.at[b, pl.ds(j * rep, rep), :],
                sem_w.at[b * n_wr + j]).wait()


def kernel(z_H, z_L, W_h, W_l, codebook, codebook_keys, W1, b1, W2, b2):
    b_sz, s_seq, hd = z_H.shape
    ld = z_L.shape[2]

    s_half = s_seq // 2
    red_ch = min(512, s_half)
    n_c = s_half // red_ch
    n_red = b_sz * n_c

    def idx_lo(t):
        tc = jnp.minimum(t, n_red - 1)
        return (tc // n_c, tc % n_c, 0)

    def idx_hi(t):
        tc = jnp.minimum(t, n_red - 1)
        return (tc // n_c, n_c + tc % n_c, 0)

    conf, idx = pl.pallas_call(
        functools.partial(_fused_body, s_seq=s_seq, n_red=n_red),
        grid=(n_red + 1,),
        in_specs=[
            pl.BlockSpec((1, red_ch, hd), idx_lo),
            pl.BlockSpec((1, red_ch, hd), idx_hi),
            pl.BlockSpec((1, red_ch, ld), idx_lo),
            pl.BlockSpec((1, red_ch, ld), idx_hi),
            pl.BlockSpec((W_h.shape[0], hd), lambda t: (0, 0)),
            pl.BlockSpec((W_l.shape[0], ld), lambda t: (0, 0)),
            pl.BlockSpec(codebook_keys.shape, lambda t: (0, 0)),
            pl.BlockSpec(W1.shape, lambda t: (0, 0)),
            pl.BlockSpec(b1.shape, lambda t: (0,)),
            pl.BlockSpec(W2.shape, lambda t: (0, 0)),
            pl.BlockSpec(b2.shape, lambda t: (0,)),
        ],
        out_specs=[
            pl.BlockSpec((b_sz,), lambda t: (0,)),
            pl.BlockSpec((b_sz,), lambda t: (0,)),
        ],
        out_shape=[
            jax.ShapeDtypeStruct((b_sz,), jnp.float32),
            jax.ShapeDtypeStruct((b_sz,), jnp.int32),
        ],
        scratch_shapes=[
            pltpu.VMEM((b_sz, hd), jnp.float32),
            pltpu.VMEM((b_sz, ld), jnp.float32),
        ],
    )(z_H, z_H, z_L, z_L, W_h, W_l, codebook_keys, W1, b1, W2, b2)

    rep = min(256, s_seq)
    n_wr = s_seq // rep
    nearest_code = pl.pallas_call(
        functools.partial(_bcast_body, rep=rep, n_wr=n_wr),
        grid_spec=pltpu.PrefetchScalarGridSpec(
            num_scalar_prefetch=1,
            grid=(1,),
            in_specs=[pl.BlockSpec(memory_space=pl.ANY)],
            out_specs=pl.BlockSpec(memory_space=pl.ANY),
            scratch_shapes=[
                pltpu.VMEM((b_sz, 1, ld), jnp.float32),
                pltpu.VMEM((b_sz, rep, ld), jnp.float32),
                pltpu.SemaphoreType.DMA((b_sz,)),
                pltpu.SemaphoreType.DMA((b_sz * n_wr,)),
            ],
        ),
        out_shape=jax.ShapeDtypeStruct((b_sz, s_seq, ld), jnp.float32),
    )(idx, codebook)

    return conf, nearest_code, idx
